# baseline (speedup 1.0000x reference)
import functools

import jax
import jax.numpy as jnp
from jax import lax
from jax.experimental import pallas as pl
from jax.experimental.pallas import tpu as pltpu
from jax.experimental.pallas import tpu_sc as plsc

NC = 2
NS = 16
NW = NC * NS
FPT = 4

_i32 = jnp.int32
_f32 = jnp.float32


def _iota16():
    return lax.iota(_i32, 16)


def _sc_hist_body(Npad, E, dst_hbm, hist_hbm, dbuf, hcnt):
    wid = lax.axis_index("s") * NC + lax.axis_index("c")
    per = E // NW
    base = wid * per

    @plsc.parallel_loop(0, Npad // 16)
    def _zero(i):
        hcnt[pl.ds(i * 16, 16)] = jnp.zeros((16,), _i32)

    pltpu.sync_copy(dst_hbm.at[pl.ds(base, per)], dbuf.at[pl.ds(0, per)])

    ones = jnp.ones((16,), _i32)
    iot = _iota16()

    @plsc.parallel_loop(0, per // 16, unroll=2)
    def _blk(b):
        ev = jnp.full((16,), b * 16, _i32) + iot
        dv = plsc.load_gather(dbuf, [ev])
        for j in range(16):
            plsc.addupdate_scatter(hcnt, [dv], ones, mask=iot == j)

    pltpu.sync_copy(hcnt, hist_hbm.at[pl.ds(wid * Npad, Npad)])


def _sc_hist(dst, Npad):
    E = dst.shape[0]
    mesh = plsc.VectorSubcoreMesh(core_axis_name="c", subcore_axis_name="s",
                                  num_cores=NC, num_subcores=NS)
    body = functools.partial(_sc_hist_body, Npad, E)
    hist = pl.kernel(
        body,
        out_type=jax.ShapeDtypeStruct((NW * Npad,), _i32),
        mesh=mesh,
        compiler_params=pltpu.CompilerParams(needs_layout_passes=False),
        scratch_types=[
            pltpu.VMEM((((E // NW) + 127) // 128 * 128,), _i32),
            pltpu.VMEM((Npad,), _i32),
        ],
    )(dst)
    return hist.reshape(NW, Npad)


def _sc_scatter_body(Npad, E, CH, xws_hbm, pk_hbm, acc_hbm, xv, av, av2, pbuf):
    wid = lax.axis_index("s") * NC + lax.axis_index("c")
    seg = Npad * FPT
    pltpu.sync_copy(xws_hbm.at[pl.ds(wid * seg, seg)], xv)

    @plsc.parallel_loop(0, seg // 16)
    def _zero(i):
        av[pl.ds(i * 16, 16)] = jnp.zeros((16,), _f32)
        av2[pl.ds(i * 16, 16)] = jnp.zeros((16,), _f32)

    iot = _iota16()
    fpat = iot % FPT
    grp = iot // FPT
    masks = [grp == j for j in range(FPT)]
    srcsel = jnp.full((16,), ((1 << 14) - 1) << 2, _i32)
    dstmask = jnp.full((16,), (1 << 14) - 1, _i32)

    def chunk(c, _):
        pltpu.sync_copy(pk_hbm.at[pl.ds(c * CH, CH)], pbuf)

        @plsc.parallel_loop(0, CH // 16, unroll=4)
        def _blk(b):
            for g in range(4):
                ev = jnp.full((16,), b * 16 + 4 * g, _i32) + grp
                pv = plsc.load_gather(pbuf, [ev])
                gi = (lax.shift_right_logical(pv, 12) & srcsel) | fpat
                si = lax.shift_left(pv & dstmask, 2) | fpat
                v = plsc.load_gather(xv, [gi])
                for j in range(FPT):
                    plsc.addupdate_scatter(av if j % 2 == 0 else av2,
                                           [si], v, mask=masks[j])

        return _

    lax.fori_loop(0, E // CH, chunk, 0)

    @plsc.parallel_loop(0, seg // 16)
    def _merge(i):
        sl = pl.ds(i * 16, 16)
        av[sl] = av[sl] + av2[sl]

    pltpu.sync_copy(av, acc_hbm.at[pl.ds(wid * seg, seg)])


def _sc_scatter(xws, pk, Npad):
    E = pk.shape[0]
    CH = 8000
    assert E % CH == 0 and CH % 16 == 0
    xws_gm = xws.reshape(Npad, NW, FPT).swapaxes(0, 1).reshape(-1)
    mesh = plsc.VectorSubcoreMesh(core_axis_name="c", subcore_axis_name="s",
                                  num_cores=NC, num_subcores=NS)
    body = functools.partial(_sc_scatter_body, Npad, E, CH)
    acc_gm = pl.kernel(
        body,
        out_type=jax.ShapeDtypeStruct((NW * Npad * FPT,), _f32),
        mesh=mesh,
        compiler_params=pltpu.CompilerParams(needs_layout_passes=False),
        scratch_types=[
            pltpu.VMEM((Npad * FPT,), _f32),
            pltpu.VMEM((Npad * FPT,), _f32),
            pltpu.VMEM((Npad * FPT,), _f32),
            pltpu.VMEM((CH,), _i32),
        ],
    )(xws_gm, pk)
    return acc_gm.reshape(NW, Npad, FPT).swapaxes(0, 1).reshape(Npad, 128)


def _tc_a_body(x_ref, hist_ref, Wn_ref, bn_ref, Wg_ref, xws_ref, dinv_ref):
    deg = 1.0 + jnp.sum(hist_ref[...], axis=0).astype(_f32)
    dinv = lax.rsqrt(deg)
    h = jnp.dot(x_ref[...], Wn_ref[...], preferred_element_type=_f32)
    h = h + bn_ref[...].reshape(1, -1)
    xw = jnp.dot(h, Wg_ref[...], preferred_element_type=_f32)
    xws_ref[...] = xw * dinv[:, None]
    dinv_ref[...] = dinv


def _tc_a(xp, hist, W_node, b_node, W_g0, Npad):
    BN = 512
    grid = (Npad // BN,)
    return pl.pallas_call(
        _tc_a_body,
        grid=grid,
        in_specs=[
            pl.BlockSpec((BN, 128), lambda i: (i, 0)),
            pl.BlockSpec((NW, BN), lambda i: (0, i)),
            pl.BlockSpec((128, 128), lambda i: (0, 0)),
            pl.BlockSpec((128,), lambda i: (0,)),
            pl.BlockSpec((128, 128), lambda i: (0, 0)),
        ],
        out_specs=[
            pl.BlockSpec((BN, 128), lambda i: (i, 0)),
            pl.BlockSpec((BN,), lambda i: (i,)),
        ],
        out_shape=[
            jax.ShapeDtypeStruct((Npad, 128), _f32),
            jax.ShapeDtypeStruct((Npad,), _f32),
        ],
    )(xp, hist, W_node, b_node, W_g0)


def _tc_b_body(acc_ref, xws_ref, dinv_ref, b_ref, W_ref, out_ref):
    dinv = dinv_ref[...]
    h = jax.nn.relu(dinv[:, None] * (acc_ref[...] + xws_ref[...])
                    + b_ref[...].reshape(1, -1))
    xw = jnp.dot(h, W_ref[...], preferred_element_type=_f32)
    out_ref[...] = xw * dinv[:, None]


def _tc_b(acc, xws, dinv, b_prev, W_next, Npad):
    BN = 512
    grid = (Npad // BN,)
    return pl.pallas_call(
        _tc_b_body,
        grid=grid,
        in_specs=[
            pl.BlockSpec((BN, 128), lambda i: (i, 0)),
            pl.BlockSpec((BN, 128), lambda i: (i, 0)),
            pl.BlockSpec((BN,), lambda i: (i,)),
            pl.BlockSpec((128,), lambda i: (0,)),
            pl.BlockSpec((128, 128), lambda i: (0, 0)),
        ],
        out_specs=pl.BlockSpec((BN, 128), lambda i: (i, 0)),
        out_shape=jax.ShapeDtypeStruct((Npad, 128), _f32),
    )(acc, xws, dinv, b_prev, W_next)


def _tc_c_body(N, G, acc_ref, xws_ref, dinv_ref, b_ref, batch_ref,
               Wa0_ref, ba0_ref, Wa1_ref, ba1_ref,
               Wo1_ref, bo1_ref, Wo2_ref, bo2_ref, out_ref):
    Npad = acc_ref.shape[0]
    dinv = dinv_ref[...]
    h = jax.nn.relu(dinv[:, None] * (acc_ref[...] + xws_ref[...])
                    + b_ref[...].reshape(1, -1))

    rowmask = (lax.broadcasted_iota(_i32, (Npad, 1), 0) < N)
    neg = jnp.float32(-1e30)

    seg = batch_ref[...].reshape(Npad, 1)
    B = (seg == lax.broadcasted_iota(_i32, (Npad, G), 1)).astype(_f32)

    def gsoftmax(logits):
        lg = jnp.where(rowmask, logits, neg)
        m = jnp.max(lg, axis=0, keepdims=True)
        e = jnp.where(rowmask, jnp.exp(lg - m), 0.0)
        return e / jnp.sum(e, axis=0, keepdims=True)

    s0 = jnp.dot(h, Wa0_ref[...], preferred_element_type=_f32) \
        + ba0_ref[...].reshape(1, 1)
    a0 = gsoftmax(s0)

    ge = lax.dot_general(B, h * a0, (((0,), (0,)), ((), ())),
                         preferred_element_type=_f32)
    P = jnp.dot(B, ge, preferred_element_type=_f32)
    h2 = h + P
    attn = h + 2.0 * P

    s1 = jnp.dot(attn, Wa1_ref[...], preferred_element_type=_f32) \
        + ba1_ref[...].reshape(1, 1)
    a1 = gsoftmax(s1)
    ge2 = lax.dot_general(B, h2 * a1, (((0,), (0,)), ((), ())),
                          preferred_element_type=_f32)

    t = jax.nn.relu(jnp.dot(ge2, Wo1_ref[...], preferred_element_type=_f32)
                    + bo1_ref[...].reshape(1, -1))
    out_ref[...] = jnp.dot(t, Wo2_ref[...], preferred_element_type=_f32) \
        + bo2_ref[...].reshape(1, 1)


def _tc_c(acc, xws, dinv, b_g2, batchp, Wa0, ba0, Wa1, ba1,
          Wo1, bo1, Wo2, bo2, N, G, Npad):
    body = functools.partial(_tc_c_body, N, G)
    return pl.pallas_call(
        body,
        out_shape=jax.ShapeDtypeStruct((G, 1), _f32),
    )(acc, xws, dinv, b_g2, batchp, Wa0, ba0, Wa1, ba1, Wo1, bo1, Wo2, bo2)


def kernel(x, edge_index, edge_attr, batch, W_node, b_node, W_edge, b_edge,
           W_g0, b_g0, W_g1, b_g1, W_g2, b_g2,
           W_a0, b_a0, W_a1, b_a1, W_o1, b_o1, W_o2, b_o2):
    N = x.shape[0]
    G = 64
    Npad = ((N + 1023) // 1024) * 1024

    src = edge_index[0]
    dst = edge_index[1]
    pk = jnp.bitwise_or(jnp.left_shift(src, 14), dst)

    xp = jnp.pad(x, ((0, Npad - N), (0, 0)))
    batchp = jnp.pad(batch, (0, Npad - N), constant_values=G)

    hist = _sc_hist(dst, Npad)
    xws, dinv = _tc_a(xp, hist, W_node, b_node, W_g0, Npad)

    acc = _sc_scatter(xws, pk, Npad)
    xws = _tc_b(acc, xws, dinv, b_g0, W_g1, Npad)

    acc = _sc_scatter(xws, pk, Npad)
    xws = _tc_b(acc, xws, dinv, b_g1, W_g2, Npad)

    acc = _sc_scatter(xws, pk, Npad)
    return _tc_c(acc, xws, dinv, b_g2, batchp, W_a0, b_a0, W_a1, b_a1,
                 W_o1, b_o1, W_o2, b_o2, N, G, Npad)


# speedup vs baseline: 1.1797x; 1.1797x over previous
import functools

import jax
import jax.numpy as jnp
from jax import lax
from jax.experimental import pallas as pl
from jax.experimental.pallas import tpu as pltpu
from jax.experimental.pallas import tpu_sc as plsc

NC = 2
NS = 16
NW = NC * NS
FPT = 4

_i32 = jnp.int32
_f32 = jnp.float32


def _iota16():
    return lax.iota(_i32, 16)


def _sc_hist_body(Npad, E, dst_hbm, hist_hbm, dbuf, hcnt):
    wid = lax.axis_index("s") * NC + lax.axis_index("c")
    per = E // NW
    base = wid * per

    @plsc.parallel_loop(0, Npad // 16)
    def _zero(i):
        hcnt[pl.ds(i * 16, 16)] = jnp.zeros((16,), _i32)

    pltpu.sync_copy(dst_hbm.at[pl.ds(base, per)], dbuf.at[pl.ds(0, per)])

    ones = jnp.ones((16,), _i32)
    iot = _iota16()

    @plsc.parallel_loop(0, per // 16, unroll=2)
    def _blk(b):
        ev = jnp.full((16,), b * 16, _i32) + iot
        dv = plsc.load_gather(dbuf, [ev])
        plsc.addupdate_scatter(hcnt, [dv], ones)

    pltpu.sync_copy(hcnt, hist_hbm.at[pl.ds(wid * Npad, Npad)])


def _sc_hist(dst, Npad):
    E = dst.shape[0]
    mesh = plsc.VectorSubcoreMesh(core_axis_name="c", subcore_axis_name="s",
                                  num_cores=NC, num_subcores=NS)
    body = functools.partial(_sc_hist_body, Npad, E)
    hist = pl.kernel(
        body,
        out_type=jax.ShapeDtypeStruct((NW * Npad,), _i32),
        mesh=mesh,
        compiler_params=pltpu.CompilerParams(needs_layout_passes=False),
        scratch_types=[
            pltpu.VMEM((((E // NW) + 127) // 128 * 128,), _i32),
            pltpu.VMEM((Npad,), _i32),
        ],
    )(dst)
    return hist.reshape(NW, Npad)


def _sc_scatter_body(Npad, E, CH, xws_hbm, pk_hbm, acc_hbm, xv, av, pbuf):
    wid = lax.axis_index("s") * NC + lax.axis_index("c")
    seg = Npad * FPT
    pltpu.sync_copy(xws_hbm.at[pl.ds(wid * seg, seg)], xv)

    @plsc.parallel_loop(0, seg // 16)
    def _zero(i):
        av[pl.ds(i * 16, 16)] = jnp.zeros((16,), _f32)

    iot = _iota16()
    fpat = iot % FPT
    grp = iot // FPT
    masks = [grp == j for j in range(FPT)]
    srcsel = jnp.full((16,), ((1 << 14) - 1) << 2, _i32)
    dstmask = jnp.full((16,), (1 << 14) - 1, _i32)

    def chunk(c, _):
        pltpu.sync_copy(pk_hbm.at[pl.ds(c * CH, CH)], pbuf)

        @plsc.parallel_loop(0, CH // 16, unroll=4)
        def _blk(b):
            for g in range(4):
                ev = jnp.full((16,), b * 16 + 4 * g, _i32) + grp
                pv = plsc.load_gather(pbuf, [ev])
                gi = (lax.shift_right_logical(pv, 12) & srcsel) | fpat
                si = lax.shift_left(pv & dstmask, 2) | fpat
                v = plsc.load_gather(xv, [gi])
                plsc.addupdate_scatter(av, [si], v)

        return _

    lax.fori_loop(0, E // CH, chunk, 0)
    pltpu.sync_copy(av, acc_hbm.at[pl.ds(wid * seg, seg)])


def _sc_scatter(xws, pk, Npad):
    E = pk.shape[0]
    CH = 32000
    assert E % CH == 0 and CH % 16 == 0
    xws_gm = xws.reshape(Npad, NW, FPT).swapaxes(0, 1).reshape(-1)
    mesh = plsc.VectorSubcoreMesh(core_axis_name="c", subcore_axis_name="s",
                                  num_cores=NC, num_subcores=NS)
    body = functools.partial(_sc_scatter_body, Npad, E, CH)
    acc_gm = pl.kernel(
        body,
        out_type=jax.ShapeDtypeStruct((NW * Npad * FPT,), _f32),
        mesh=mesh,
        compiler_params=pltpu.CompilerParams(needs_layout_passes=False),
        scratch_types=[
            pltpu.VMEM((Npad * FPT,), _f32),
            pltpu.VMEM((Npad * FPT,), _f32),
            pltpu.VMEM((CH,), _i32),
        ],
    )(xws_gm, pk)
    return acc_gm.reshape(NW, Npad, FPT).swapaxes(0, 1).reshape(Npad, 128)


def _tc_a_body(x_ref, hist_ref, Wn_ref, bn_ref, Wg_ref, xws_ref, dinv_ref):
    deg = 1.0 + jnp.sum(hist_ref[...], axis=0).astype(_f32)
    dinv = lax.rsqrt(deg)
    h = jnp.dot(x_ref[...], Wn_ref[...], preferred_element_type=_f32)
    h = h + bn_ref[...].reshape(1, -1)
    xw = jnp.dot(h, Wg_ref[...], preferred_element_type=_f32)
    xws_ref[...] = xw * dinv[:, None]
    dinv_ref[...] = dinv


def _tc_a(xp, hist, W_node, b_node, W_g0, Npad):
    BN = 512
    grid = (Npad // BN,)
    return pl.pallas_call(
        _tc_a_body,
        grid=grid,
        in_specs=[
            pl.BlockSpec((BN, 128), lambda i: (i, 0)),
            pl.BlockSpec((NW, BN), lambda i: (0, i)),
            pl.BlockSpec((128, 128), lambda i: (0, 0)),
            pl.BlockSpec((128,), lambda i: (0,)),
            pl.BlockSpec((128, 128), lambda i: (0, 0)),
        ],
        out_specs=[
            pl.BlockSpec((BN, 128), lambda i: (i, 0)),
            pl.BlockSpec((BN,), lambda i: (i,)),
        ],
        out_shape=[
            jax.ShapeDtypeStruct((Npad, 128), _f32),
            jax.ShapeDtypeStruct((Npad,), _f32),
        ],
    )(xp, hist, W_node, b_node, W_g0)


def _tc_b_body(acc_ref, xws_ref, dinv_ref, b_ref, W_ref, out_ref):
    dinv = dinv_ref[...]
    h = jax.nn.relu(dinv[:, None] * (acc_ref[...] + xws_ref[...])
                    + b_ref[...].reshape(1, -1))
    xw = jnp.dot(h, W_ref[...], preferred_element_type=_f32)
    out_ref[...] = xw * dinv[:, None]


def _tc_b(acc, xws, dinv, b_prev, W_next, Npad):
    BN = 512
    grid = (Npad // BN,)
    return pl.pallas_call(
        _tc_b_body,
        grid=grid,
        in_specs=[
            pl.BlockSpec((BN, 128), lambda i: (i, 0)),
            pl.BlockSpec((BN, 128), lambda i: (i, 0)),
            pl.BlockSpec((BN,), lambda i: (i,)),
            pl.BlockSpec((128,), lambda i: (0,)),
            pl.BlockSpec((128, 128), lambda i: (0, 0)),
        ],
        out_specs=pl.BlockSpec((BN, 128), lambda i: (i, 0)),
        out_shape=jax.ShapeDtypeStruct((Npad, 128), _f32),
    )(acc, xws, dinv, b_prev, W_next)


def _tc_c_body(N, G, acc_ref, xws_ref, dinv_ref, b_ref, batch_ref,
               Wa0_ref, ba0_ref, Wa1_ref, ba1_ref,
               Wo1_ref, bo1_ref, Wo2_ref, bo2_ref, out_ref):
    Npad = acc_ref.shape[0]
    dinv = dinv_ref[...]
    h = jax.nn.relu(dinv[:, None] * (acc_ref[...] + xws_ref[...])
                    + b_ref[...].reshape(1, -1))

    rowmask = (lax.broadcasted_iota(_i32, (Npad, 1), 0) < N)
    neg = jnp.float32(-1e30)

    seg = batch_ref[...].reshape(Npad, 1)
    B = (seg == lax.broadcasted_iota(_i32, (Npad, G), 1)).astype(_f32)

    def gsoftmax(logits):
        lg = jnp.where(rowmask, logits, neg)
        m = jnp.max(lg, axis=0, keepdims=True)
        e = jnp.where(rowmask, jnp.exp(lg - m), 0.0)
        return e / jnp.sum(e, axis=0, keepdims=True)

    s0 = jnp.dot(h, Wa0_ref[...], preferred_element_type=_f32) \
        + ba0_ref[...].reshape(1, 1)
    a0 = gsoftmax(s0)

    ge = lax.dot_general(B, h * a0, (((0,), (0,)), ((), ())),
                         preferred_element_type=_f32)
    P = jnp.dot(B, ge, preferred_element_type=_f32)
    h2 = h + P
    attn = h + 2.0 * P

    s1 = jnp.dot(attn, Wa1_ref[...], preferred_element_type=_f32) \
        + ba1_ref[...].reshape(1, 1)
    a1 = gsoftmax(s1)
    ge2 = lax.dot_general(B, h2 * a1, (((0,), (0,)), ((), ())),
                          preferred_element_type=_f32)

    t = jax.nn.relu(jnp.dot(ge2, Wo1_ref[...], preferred_element_type=_f32)
                    + bo1_ref[...].reshape(1, -1))
    out_ref[...] = jnp.dot(t, Wo2_ref[...], preferred_element_type=_f32) \
        + bo2_ref[...].reshape(1, 1)


def _tc_c(acc, xws, dinv, b_g2, batchp, Wa0, ba0, Wa1, ba1,
          Wo1, bo1, Wo2, bo2, N, G, Npad):
    body = functools.partial(_tc_c_body, N, G)
    return pl.pallas_call(
        body,
        out_shape=jax.ShapeDtypeStruct((G, 1), _f32),
    )(acc, xws, dinv, b_g2, batchp, Wa0, ba0, Wa1, ba1, Wo1, bo1, Wo2, bo2)


def kernel(x, edge_index, edge_attr, batch, W_node, b_node, W_edge, b_edge,
           W_g0, b_g0, W_g1, b_g1, W_g2, b_g2,
           W_a0, b_a0, W_a1, b_a1, W_o1, b_o1, W_o2, b_o2):
    N = x.shape[0]
    G = 64
    Npad = ((N + 1023) // 1024) * 1024

    src = edge_index[0]
    dst = edge_index[1]
    pk = jnp.bitwise_or(jnp.left_shift(src, 14), dst)

    xp = jnp.pad(x, ((0, Npad - N), (0, 0)))
    batchp = jnp.pad(batch, (0, Npad - N), constant_values=G)

    hist = _sc_hist(dst, Npad)
    xws, dinv = _tc_a(xp, hist, W_node, b_node, W_g0, Npad)

    acc = _sc_scatter(xws, pk, Npad)
    xws = _tc_b(acc, xws, dinv, b_g0, W_g1, Npad)

    acc = _sc_scatter(xws, pk, Npad)
    xws = _tc_b(acc, xws, dinv, b_g1, W_g2, Npad)

    acc = _sc_scatter(xws, pk, Npad)
    return _tc_c(acc, xws, dinv, b_g2, batchp, W_a0, b_a0, W_a1, b_a1,
                 W_o1, b_o1, W_o2, b_o2, N, G, Npad)


# speedup vs baseline: 1.2371x; 1.0486x over previous
import functools

import jax
import jax.numpy as jnp
from jax import lax
from jax.experimental import pallas as pl
from jax.experimental.pallas import tpu as pltpu
from jax.experimental.pallas import tpu_sc as plsc

NC = 2
NS = 16
NW = NC * NS
FPT = 4

_i32 = jnp.int32
_f32 = jnp.float32


def _iota16():
    return lax.iota(_i32, 16)


def _sc_hist_body(Npad, E, dst_hbm, hist_hbm, dbuf, hcnt):
    wid = lax.axis_index("s") * NC + lax.axis_index("c")
    per = E // NW
    base = wid * per

    @plsc.parallel_loop(0, Npad // 16)
    def _zero(i):
        hcnt[pl.ds(i * 16, 16)] = jnp.zeros((16,), _i32)

    pltpu.sync_copy(dst_hbm.at[pl.ds(base, per)], dbuf.at[pl.ds(0, per)])

    ones = jnp.ones((16,), _i32)
    iot = _iota16()

    @plsc.parallel_loop(0, per // 16, unroll=2)
    def _blk(b):
        ev = jnp.full((16,), b * 16, _i32) + iot
        dv = plsc.load_gather(dbuf, [ev])
        plsc.addupdate_scatter(hcnt, [dv], ones)

    pltpu.sync_copy(hcnt, hist_hbm.at[pl.ds(wid * Npad, Npad)])


def _sc_hist(dst, Npad):
    E = dst.shape[0]
    mesh = plsc.VectorSubcoreMesh(core_axis_name="c", subcore_axis_name="s",
                                  num_cores=NC, num_subcores=NS)
    body = functools.partial(_sc_hist_body, Npad, E)
    hist = pl.kernel(
        body,
        out_type=jax.ShapeDtypeStruct((NW * Npad,), _i32),
        mesh=mesh,
        compiler_params=pltpu.CompilerParams(needs_layout_passes=False),
        scratch_types=[
            pltpu.VMEM((((E // NW) + 127) // 128 * 128,), _i32),
            pltpu.VMEM((Npad,), _i32),
        ],
    )(dst)
    return hist.reshape(NW, Npad)


def _sc_scatter_body(Npad, E, CH, xws_hbm, pk_hbm, acc_hbm, xv, av, pbuf):
    wid = lax.axis_index("s") * NC + lax.axis_index("c")
    seg = Npad * FPT
    pltpu.sync_copy(xws_hbm.at[pl.ds(wid * seg, seg)], xv)

    @plsc.parallel_loop(0, seg // 16)
    def _zero(i):
        av[pl.ds(i * 16, 16)] = jnp.zeros((16,), _f32)

    iot = _iota16()
    fpat = iot % FPT
    grp = iot // FPT
    masks = [grp == j for j in range(FPT)]
    srcsel = jnp.full((16,), ((1 << 14) - 1) << 2, _i32)
    dstmask = jnp.full((16,), (1 << 14) - 1, _i32)

    def chunk(c, _):
        pltpu.sync_copy(pk_hbm.at[pl.ds(c * CH, CH)], pbuf)

        @plsc.parallel_loop(0, CH // 16, unroll=4)
        def _blk(b):
            pv16 = pbuf[pl.ds(b * 16, 16)]
            gib = lax.shift_right_logical(pv16, 12) & srcsel
            sib = lax.shift_left(pv16 & dstmask, 2)
            for g in range(4):
                patg = grp + 4 * g
                gi = jnp.take_along_axis(gib, patg, axis=0) | fpat
                si = jnp.take_along_axis(sib, patg, axis=0) | fpat
                v = plsc.load_gather(xv, [gi])
                plsc.addupdate_scatter(av, [si], v)

        return _

    lax.fori_loop(0, E // CH, chunk, 0)
    pltpu.sync_copy(av, acc_hbm.at[pl.ds(wid * seg, seg)])


def _sc_scatter(xws, pk, Npad):
    E = pk.shape[0]
    CH = 32000
    assert E % CH == 0 and CH % 16 == 0
    xws_gm = xws.reshape(Npad, NW, FPT).swapaxes(0, 1).reshape(-1)
    mesh = plsc.VectorSubcoreMesh(core_axis_name="c", subcore_axis_name="s",
                                  num_cores=NC, num_subcores=NS)
    body = functools.partial(_sc_scatter_body, Npad, E, CH)
    acc_gm = pl.kernel(
        body,
        out_type=jax.ShapeDtypeStruct((NW * Npad * FPT,), _f32),
        mesh=mesh,
        compiler_params=pltpu.CompilerParams(needs_layout_passes=False),
        scratch_types=[
            pltpu.VMEM((Npad * FPT,), _f32),
            pltpu.VMEM((Npad * FPT,), _f32),
            pltpu.VMEM((CH,), _i32),
        ],
    )(xws_gm, pk)
    return acc_gm.reshape(NW, Npad, FPT).swapaxes(0, 1).reshape(Npad, 128)


def _tc_a_body(x_ref, hist_ref, Wn_ref, bn_ref, Wg_ref, xws_ref, dinv_ref):
    deg = 1.0 + jnp.sum(hist_ref[...], axis=0).astype(_f32)
    dinv = lax.rsqrt(deg)
    h = jnp.dot(x_ref[...], Wn_ref[...], preferred_element_type=_f32)
    h = h + bn_ref[...].reshape(1, -1)
    xw = jnp.dot(h, Wg_ref[...], preferred_element_type=_f32)
    xws_ref[...] = xw * dinv[:, None]
    dinv_ref[...] = dinv


def _tc_a(xp, hist, W_node, b_node, W_g0, Npad):
    BN = 512
    grid = (Npad // BN,)
    return pl.pallas_call(
        _tc_a_body,
        grid=grid,
        in_specs=[
            pl.BlockSpec((BN, 128), lambda i: (i, 0)),
            pl.BlockSpec((NW, BN), lambda i: (0, i)),
            pl.BlockSpec((128, 128), lambda i: (0, 0)),
            pl.BlockSpec((128,), lambda i: (0,)),
            pl.BlockSpec((128, 128), lambda i: (0, 0)),
        ],
        out_specs=[
            pl.BlockSpec((BN, 128), lambda i: (i, 0)),
            pl.BlockSpec((BN,), lambda i: (i,)),
        ],
        out_shape=[
            jax.ShapeDtypeStruct((Npad, 128), _f32),
            jax.ShapeDtypeStruct((Npad,), _f32),
        ],
    )(xp, hist, W_node, b_node, W_g0)


def _tc_b_body(acc_ref, xws_ref, dinv_ref, b_ref, W_ref, out_ref):
    dinv = dinv_ref[...]
    h = jax.nn.relu(dinv[:, None] * (acc_ref[...] + xws_ref[...])
                    + b_ref[...].reshape(1, -1))
    xw = jnp.dot(h, W_ref[...], preferred_element_type=_f32)
    out_ref[...] = xw * dinv[:, None]


def _tc_b(acc, xws, dinv, b_prev, W_next, Npad):
    BN = 512
    grid = (Npad // BN,)
    return pl.pallas_call(
        _tc_b_body,
        grid=grid,
        in_specs=[
            pl.BlockSpec((BN, 128), lambda i: (i, 0)),
            pl.BlockSpec((BN, 128), lambda i: (i, 0)),
            pl.BlockSpec((BN,), lambda i: (i,)),
            pl.BlockSpec((128,), lambda i: (0,)),
            pl.BlockSpec((128, 128), lambda i: (0, 0)),
        ],
        out_specs=pl.BlockSpec((BN, 128), lambda i: (i, 0)),
        out_shape=jax.ShapeDtypeStruct((Npad, 128), _f32),
    )(acc, xws, dinv, b_prev, W_next)


def _tc_c_body(N, G, acc_ref, xws_ref, dinv_ref, b_ref, batch_ref,
               Wa0_ref, ba0_ref, Wa1_ref, ba1_ref,
               Wo1_ref, bo1_ref, Wo2_ref, bo2_ref, out_ref):
    Npad = acc_ref.shape[0]
    dinv = dinv_ref[...]
    h = jax.nn.relu(dinv[:, None] * (acc_ref[...] + xws_ref[...])
                    + b_ref[...].reshape(1, -1))

    rowmask = (lax.broadcasted_iota(_i32, (Npad, 1), 0) < N)
    neg = jnp.float32(-1e30)

    seg = batch_ref[...].reshape(Npad, 1)
    B = (seg == lax.broadcasted_iota(_i32, (Npad, G), 1)).astype(_f32)

    def gsoftmax(logits):
        lg = jnp.where(rowmask, logits, neg)
        m = jnp.max(lg, axis=0, keepdims=True)
        e = jnp.where(rowmask, jnp.exp(lg - m), 0.0)
        return e / jnp.sum(e, axis=0, keepdims=True)

    s0 = jnp.dot(h, Wa0_ref[...], preferred_element_type=_f32) \
        + ba0_ref[...].reshape(1, 1)
    a0 = gsoftmax(s0)

    ge = lax.dot_general(B, h * a0, (((0,), (0,)), ((), ())),
                         preferred_element_type=_f32)
    P = jnp.dot(B, ge, preferred_element_type=_f32)
    h2 = h + P
    attn = h + 2.0 * P

    s1 = jnp.dot(attn, Wa1_ref[...], preferred_element_type=_f32) \
        + ba1_ref[...].reshape(1, 1)
    a1 = gsoftmax(s1)
    ge2 = lax.dot_general(B, h2 * a1, (((0,), (0,)), ((), ())),
                          preferred_element_type=_f32)

    t = jax.nn.relu(jnp.dot(ge2, Wo1_ref[...], preferred_element_type=_f32)
                    + bo1_ref[...].reshape(1, -1))
    out_ref[...] = jnp.dot(t, Wo2_ref[...], preferred_element_type=_f32) \
        + bo2_ref[...].reshape(1, 1)


def _tc_c(acc, xws, dinv, b_g2, batchp, Wa0, ba0, Wa1, ba1,
          Wo1, bo1, Wo2, bo2, N, G, Npad):
    body = functools.partial(_tc_c_body, N, G)
    return pl.pallas_call(
        body,
        out_shape=jax.ShapeDtypeStruct((G, 1), _f32),
    )(acc, xws, dinv, b_g2, batchp, Wa0, ba0, Wa1, ba1, Wo1, bo1, Wo2, bo2)


def kernel(x, edge_index, edge_attr, batch, W_node, b_node, W_edge, b_edge,
           W_g0, b_g0, W_g1, b_g1, W_g2, b_g2,
           W_a0, b_a0, W_a1, b_a1, W_o1, b_o1, W_o2, b_o2):
    N = x.shape[0]
    G = 64
    Npad = ((N + 1023) // 1024) * 1024

    src = edge_index[0]
    dst = edge_index[1]
    pk = jnp.bitwise_or(jnp.left_shift(src, 14), dst)

    xp = jnp.pad(x, ((0, Npad - N), (0, 0)))
    batchp = jnp.pad(batch, (0, Npad - N), constant_values=G)

    hist = _sc_hist(dst, Npad)
    xws, dinv = _tc_a(xp, hist, W_node, b_node, W_g0, Npad)

    acc = _sc_scatter(xws, pk, Npad)
    xws = _tc_b(acc, xws, dinv, b_g0, W_g1, Npad)

    acc = _sc_scatter(xws, pk, Npad)
    xws = _tc_b(acc, xws, dinv, b_g1, W_g2, Npad)

    acc = _sc_scatter(xws, pk, Npad)
    return _tc_c(acc, xws, dinv, b_g2, batchp, W_a0, b_a0, W_a1, b_a1,
                 W_o1, b_o1, W_o2, b_o2, N, G, Npad)
